# baseline (speedup 1.0000x reference)
import functools

import jax
import jax.numpy as jnp
from jax import lax
from jax.experimental import pallas as pl
from jax.experimental.pallas import tpu as pltpu
from jax.experimental.pallas import tpu_sc as plsc

KT = 512
TT = 256


SK = 128


def _argmin_body(w_ref, wsq_ref, kio_ref, r_ref, rsq_ref, idx_ref,
                 best_d, best_i, *, n_k):
    k = pl.program_id(1)

    @pl.when(k == 0)
    def _init():
        best_d[...] = jnp.full((1, TT), jnp.inf, jnp.float32)
        best_i[...] = jnp.zeros((1, TT), jnp.float32)

    r_bf = r_ref[...]
    r2 = rsq_ref[0]
    m8 = None
    i8 = None
    for s in range(KT // SK):
        w_bf = w_ref[pl.ds(s * SK, SK), :]
        cross = lax.dot_general(
            w_bf, r_bf, (((1,), (1,)), ((), ())),
            preferred_element_type=jnp.float32)
        d2 = (r2 - 2.0 * cross) + wsq_ref[pl.ds(s * SK, SK), :]
        io = kio_ref[pl.ds(s * SK, SK), :]
        v = d2
        n = SK
        first = True
        while n > 8:
            h = n // 2
            a, b = v[:h], v[h:]
            lt = b < a
            if first:
                io = jnp.where(lt, io[h:], io[:h])
                first = False
            else:
                io = jnp.where(lt, io[h:], io[:h])
            v = jnp.minimum(a, b)
            n = h
        if m8 is None:
            m8, i8 = v, io
        else:
            lt = v < m8
            i8 = jnp.where(lt, io, i8)
            m8 = jnp.minimum(m8, v)
    m = jnp.min(m8, axis=0)[None, :]
    ii = jnp.min(jnp.where(m8 == m, i8, jnp.float32(2**24)),
                 axis=0)[None, :]
    bd = best_d[...]
    upd = m < bd
    best_d[...] = jnp.where(upd, m, bd)
    best_i[...] = jnp.where(upd, ii, best_i[...])

    @pl.when(k == n_k - 1)
    def _flush():
        idx_ref[...] = best_i[...].astype(jnp.int32).reshape(1, 1, TT)


def _tc_argmin(r_bf, rsq, w_bf, wsq, kio):
    N, D = r_bf.shape
    K = w_bf.shape[0]
    n_k = K // KT
    n_t = N // TT
    return pl.pallas_call(
        functools.partial(_argmin_body, n_k=n_k),
        grid=(n_t, n_k),
        in_specs=[
            pl.BlockSpec((KT, D), lambda t, k: (k, 0)),
            pl.BlockSpec((KT, 1), lambda t, k: (k, 0)),
            pl.BlockSpec((KT, 1), lambda t, k: (k, 0)),
            pl.BlockSpec((TT, D), lambda t, k: (t, 0)),
            pl.BlockSpec((1, 1, TT), lambda t, k: (t, 0, 0)),
        ],
        out_specs=pl.BlockSpec((1, 1, TT), lambda t, k: (t, 0, 0)),
        out_shape=jax.ShapeDtypeStruct((n_t, 1, TT), jnp.int32),
        scratch_shapes=[pltpu.VMEM((1, TT), jnp.float32),
                        pltpu.VMEM((1, TT), jnp.float32)],
    )(w_bf, wsq.reshape(K, 1), kio.reshape(K, 1), r_bf,
      rsq.reshape(n_t, 1, TT))


def _sc_gather_sub(table, idx, res):
    N, D = res.shape
    info = plsc.get_sparse_core_info()
    NC, NS = info.num_cores, info.num_subcores
    bpw = N // (NC * NS)
    mesh = plsc.VectorSubcoreMesh(core_axis_name="c", subcore_axis_name="s")

    def body(table_hbm, idx_hbm, res_hbm, out_hbm, idx_v, rows_v, res_v, sem):
        wid = lax.axis_index("s") * NC + lax.axis_index("c")
        base = wid * bpw
        pltpu.sync_copy(idx_hbm.at[pl.ds(base, bpw)], idx_v)
        pltpu.async_copy(table_hbm.at[idx_v], rows_v, sem).wait()
        pltpu.sync_copy(res_hbm.at[pl.ds(base, bpw)], res_v)

        def row(i, carry):
            for j in range(D // 16):
                s = pl.ds(j * 16, 16)
                res_v[i, s] = res_v[i, s] - rows_v[i, s]
            return carry

        lax.fori_loop(0, bpw, row, 0)
        pltpu.sync_copy(res_v, out_hbm.at[pl.ds(base, bpw)])

    f = pl.kernel(
        body,
        out_type=jax.ShapeDtypeStruct((N, D), jnp.float32),
        mesh=mesh,
        scratch_types=[pltpu.VMEM((bpw,), jnp.int32),
                       pltpu.VMEM((bpw, D), jnp.float32),
                       pltpu.VMEM((bpw, D), jnp.float32),
                       pltpu.SemaphoreType.DMA],
    )
    return f(table, idx, res)


def _sc_gather_sub_final(table, idx, res, x):
    N, D = res.shape
    info = plsc.get_sparse_core_info()
    NC, NS = info.num_cores, info.num_subcores
    bpw = N // (NC * NS)
    mesh = plsc.VectorSubcoreMesh(core_axis_name="c", subcore_axis_name="s")

    def body(table_hbm, idx_hbm, res_hbm, x_hbm, out_hbm,
             idx_v, rows_v, res_v, x_v, sem):
        wid = lax.axis_index("s") * NC + lax.axis_index("c")
        base = wid * bpw
        pltpu.sync_copy(idx_hbm.at[pl.ds(base, bpw)], idx_v)
        pltpu.async_copy(table_hbm.at[idx_v], rows_v, sem).wait()
        pltpu.sync_copy(res_hbm.at[pl.ds(base, bpw)], res_v)
        pltpu.sync_copy(x_hbm.at[pl.ds(base, bpw)], x_v)

        def row(i, carry):
            for j in range(D // 16):
                s = pl.ds(j * 16, 16)
                x_v[i, s] = x_v[i, s] - res_v[i, s] + rows_v[i, s]
            return carry

        lax.fori_loop(0, bpw, row, 0)
        pltpu.sync_copy(x_v, out_hbm.at[pl.ds(base, bpw)])

    f = pl.kernel(
        body,
        out_type=jax.ShapeDtypeStruct((N, D), jnp.float32),
        mesh=mesh,
        scratch_types=[pltpu.VMEM((bpw,), jnp.int32),
                       pltpu.VMEM((bpw, D), jnp.float32),
                       pltpu.VMEM((bpw, D), jnp.float32),
                       pltpu.VMEM((bpw, D), jnp.float32),
                       pltpu.SemaphoreType.DMA],
    )
    return f(table, idx, res, x)


def kernel(input, codebooks):
    B, D, T = input.shape
    L, K, _ = codebooks.shape
    x = jnp.transpose(input, (0, 2, 1)).reshape(B * T, D)
    cb_bf = codebooks.astype(jnp.bfloat16)
    wsq_all = jnp.sum(codebooks * codebooks, axis=-1)
    kio = jnp.arange(K, dtype=jnp.float32)
    res = x
    idx_list = []
    out_flat = None
    for l in range(L):
        rsq = jnp.sum(res * res, axis=-1)
        idx3 = _tc_argmin(res.astype(jnp.bfloat16), rsq,
                          cb_bf[l], wsq_all[l], kio)
        idx_flat = idx3.reshape(-1)
        idx_list.append(idx3.reshape(B, T))
        if l + 1 < L:
            res = _sc_gather_sub(codebooks[l], idx_flat, res)
        else:
            out_flat = _sc_gather_sub_final(codebooks[l], idx_flat, res, x)
    out = out_flat.reshape(B, T, D).transpose(0, 2, 1).reshape(input.shape)
    indices = jnp.stack(idx_list, axis=1).reshape((B, L, T))
    return out, indices


# speedup vs baseline: 2.1268x; 2.1268x over previous
import functools

import jax
import jax.numpy as jnp
from jax import lax
from jax.experimental import pallas as pl
from jax.experimental.pallas import tpu as pltpu
from jax.experimental.pallas import tpu_sc as plsc

KT = 512
TT = 256


SK = 128


def _argmin_body(w_ref, wsq_ref, kio_ref, r_ref, rsq_ref, idx_ref, *, n_s):
    r_bf = r_ref[...]
    r2 = rsq_ref[0]
    m8 = None
    i8 = None
    for s in range(n_s):
        w2 = w_ref[pl.ds(s * SK, SK), :]
        mcross = lax.dot_general(
            w2, r_bf, (((1,), (1,)), ((), ())),
            preferred_element_type=jnp.float32)
        d2 = (r2 + mcross) + wsq_ref[pl.ds(s * SK, SK), :]
        io = kio_ref[pl.ds(s * SK, SK), :]
        v = d2
        n = SK
        while n > 8:
            h = n // 2
            a, b = v[:h], v[h:]
            lt = b < a
            io = jnp.where(lt, io[h:], io[:h])
            v = jnp.minimum(a, b)
            n = h
        if m8 is None:
            m8, i8 = v, io
        else:
            lt = v < m8
            i8 = jnp.where(lt, io, i8)
            m8 = jnp.minimum(m8, v)
    m = jnp.min(m8, axis=0)[None, :]
    ii = jnp.min(jnp.where(m8 == m, i8, jnp.float32(2**24)),
                 axis=0)[None, :]
    idx_ref[...] = ii.astype(jnp.int32).reshape(1, 1, TT)


def _tc_argmin(r_bf, rsq, w2_bf, wsq, kio):
    N, D = r_bf.shape
    K = w2_bf.shape[0]
    n_t = N // TT
    return pl.pallas_call(
        functools.partial(_argmin_body, n_s=K // SK),
        grid=(n_t,),
        in_specs=[
            pl.BlockSpec((K, D), lambda t: (0, 0)),
            pl.BlockSpec((K, 1), lambda t: (0, 0)),
            pl.BlockSpec((K, 1), lambda t: (0, 0)),
            pl.BlockSpec((TT, D), lambda t: (t, 0)),
            pl.BlockSpec((1, 1, TT), lambda t: (t, 0, 0)),
        ],
        out_specs=pl.BlockSpec((1, 1, TT), lambda t: (t, 0, 0)),
        out_shape=jax.ShapeDtypeStruct((n_t, 1, TT), jnp.int32),
    )(w2_bf, wsq.reshape(K, 1), kio.reshape(K, 1), r_bf,
      rsq.reshape(n_t, 1, TT))


def _sc_gather_sub(table, idx, res):
    N, D = res.shape
    info = plsc.get_sparse_core_info()
    NC, NS = info.num_cores, info.num_subcores
    bpw = N // (NC * NS)
    mesh = plsc.VectorSubcoreMesh(core_axis_name="c", subcore_axis_name="s")

    def body(table_hbm, idx_hbm, res_hbm, out_hbm, idx_v, rows_v, res_v, sem):
        wid = lax.axis_index("s") * NC + lax.axis_index("c")
        base = wid * bpw
        pltpu.sync_copy(idx_hbm.at[pl.ds(base, bpw)], idx_v)
        pltpu.async_copy(table_hbm.at[idx_v], rows_v, sem).wait()
        pltpu.sync_copy(res_hbm.at[pl.ds(base, bpw)], res_v)

        def row(i, carry):
            for j in range(D // 16):
                s = pl.ds(j * 16, 16)
                res_v[i, s] = res_v[i, s] - rows_v[i, s]
            return carry

        lax.fori_loop(0, bpw, row, 0)
        pltpu.sync_copy(res_v, out_hbm.at[pl.ds(base, bpw)])

    f = pl.kernel(
        body,
        out_type=jax.ShapeDtypeStruct((N, D), jnp.float32),
        mesh=mesh,
        scratch_types=[pltpu.VMEM((bpw,), jnp.int32),
                       pltpu.VMEM((bpw, D), jnp.float32),
                       pltpu.VMEM((bpw, D), jnp.float32),
                       pltpu.SemaphoreType.DMA],
    )
    return f(table, idx, res)


def _sc_gather_sub_final(table, idx, res, x):
    N, D = res.shape
    info = plsc.get_sparse_core_info()
    NC, NS = info.num_cores, info.num_subcores
    bpw = N // (NC * NS)
    mesh = plsc.VectorSubcoreMesh(core_axis_name="c", subcore_axis_name="s")

    def body(table_hbm, idx_hbm, res_hbm, x_hbm, out_hbm,
             idx_v, rows_v, res_v, x_v, sem):
        wid = lax.axis_index("s") * NC + lax.axis_index("c")
        base = wid * bpw
        pltpu.sync_copy(idx_hbm.at[pl.ds(base, bpw)], idx_v)
        pltpu.async_copy(table_hbm.at[idx_v], rows_v, sem).wait()
        pltpu.sync_copy(res_hbm.at[pl.ds(base, bpw)], res_v)
        pltpu.sync_copy(x_hbm.at[pl.ds(base, bpw)], x_v)

        def row(i, carry):
            for j in range(D // 16):
                s = pl.ds(j * 16, 16)
                x_v[i, s] = x_v[i, s] - res_v[i, s] + rows_v[i, s]
            return carry

        lax.fori_loop(0, bpw, row, 0)
        pltpu.sync_copy(x_v, out_hbm.at[pl.ds(base, bpw)])

    f = pl.kernel(
        body,
        out_type=jax.ShapeDtypeStruct((N, D), jnp.float32),
        mesh=mesh,
        scratch_types=[pltpu.VMEM((bpw,), jnp.int32),
                       pltpu.VMEM((bpw, D), jnp.float32),
                       pltpu.VMEM((bpw, D), jnp.float32),
                       pltpu.VMEM((bpw, D), jnp.float32),
                       pltpu.SemaphoreType.DMA],
    )
    return f(table, idx, res, x)


def kernel(input, codebooks):
    B, D, T = input.shape
    L, K, _ = codebooks.shape
    x = jnp.transpose(input, (0, 2, 1)).reshape(B * T, D)
    cb2_bf = (codebooks * -2.0).astype(jnp.bfloat16)
    wsq_all = jnp.sum(codebooks * codebooks, axis=-1)
    kio = jnp.arange(K, dtype=jnp.float32)
    res = x
    idx_list = []
    out_flat = None
    for l in range(L):
        rsq = jnp.sum(res * res, axis=-1)
        idx3 = _tc_argmin(res.astype(jnp.bfloat16), rsq,
                          cb2_bf[l], wsq_all[l], kio)
        idx_flat = idx3.reshape(-1)
        idx_list.append(idx3.reshape(B, T))
        if l + 1 < L:
            res = _sc_gather_sub(codebooks[l], idx_flat, res)
        else:
            out_flat = _sc_gather_sub_final(codebooks[l], idx_flat, res, x)
    out = out_flat.reshape(B, T, D).transpose(0, 2, 1).reshape(input.shape)
    indices = jnp.stack(idx_list, axis=1).reshape((B, L, T))
    return out, indices


# speedup vs baseline: 2.2298x; 1.0484x over previous
import functools

import jax
import jax.numpy as jnp
from jax import lax
from jax.experimental import pallas as pl
from jax.experimental.pallas import tpu as pltpu
from jax.experimental.pallas import tpu_sc as plsc

KT = 512
TT = 256


SK = 128


TPS = 2


def _argmin_body(w_ref, wsq_ref, kio_ref, r_ref, rsq_ref, idx_ref, *, n_s):
    r_bfs = [r_ref[pl.ds(j * TT, TT), :].astype(jnp.bfloat16)
             for j in range(TPS)]
    r2s = [rsq_ref[j] for j in range(TPS)]
    m8 = [None] * TPS
    i8 = [None] * TPS
    for s in range(n_s):
        w2 = w_ref[pl.ds(s * SK, SK), :]
        wsq_c = wsq_ref[pl.ds(s * SK, SK), :]
        io_c = kio_ref[pl.ds(s * SK, SK), :]
        for j in range(TPS):
            mcross = lax.dot_general(
                w2, r_bfs[j], (((1,), (1,)), ((), ())),
                preferred_element_type=jnp.float32)
            d2 = (r2s[j] + mcross) + wsq_c
            io = io_c
            v = d2
            n = SK
            while n > 8:
                h = n // 2
                a, b = v[:h], v[h:]
                lt = b < a
                io = jnp.where(lt, io[h:], io[:h])
                v = jnp.minimum(a, b)
                n = h
            if m8[j] is None:
                m8[j], i8[j] = v, io
            else:
                lt = v < m8[j]
                i8[j] = jnp.where(lt, io, i8[j])
                m8[j] = jnp.minimum(m8[j], v)
    for j in range(TPS):
        m = jnp.min(m8[j], axis=0)[None, :]
        ii = jnp.min(jnp.where(m8[j] == m, i8[j], jnp.float32(2**24)),
                     axis=0)[None, :]
        idx_ref[j] = ii.astype(jnp.int32).reshape(1, TT)


def _tc_argmin(res, rsq, w2_bf, wsq, kio):
    N, D = res.shape
    K = w2_bf.shape[0]
    n_t = N // TT
    n_g = n_t // TPS
    return pl.pallas_call(
        functools.partial(_argmin_body, n_s=K // SK),
        grid=(n_g,),
        in_specs=[
            pl.BlockSpec((K, D), lambda t: (0, 0)),
            pl.BlockSpec((K, 1), lambda t: (0, 0)),
            pl.BlockSpec((K, 1), lambda t: (0, 0)),
            pl.BlockSpec((TPS * TT, D), lambda t: (t, 0)),
            pl.BlockSpec((TPS, 1, TT), lambda t: (t, 0, 0)),
        ],
        out_specs=pl.BlockSpec((TPS, 1, TT), lambda t: (t, 0, 0)),
        out_shape=jax.ShapeDtypeStruct((n_t, 1, TT), jnp.int32),
    )(w2_bf, wsq.reshape(K, 1), kio.reshape(K, 1), res,
      rsq.reshape(n_t, 1, TT))


def _sc_gather_sub(table, idx, res):
    N, D = res.shape
    info = plsc.get_sparse_core_info()
    NC, NS = info.num_cores, info.num_subcores
    bpw = N // (NC * NS)
    mesh = plsc.VectorSubcoreMesh(core_axis_name="c", subcore_axis_name="s")

    def body(table_hbm, idx_hbm, res_hbm, out_hbm, idx_v, rows_v, res_v, sem):
        wid = lax.axis_index("s") * NC + lax.axis_index("c")
        base = wid * bpw
        pltpu.sync_copy(idx_hbm.at[pl.ds(base, bpw)], idx_v)
        pltpu.async_copy(table_hbm.at[idx_v], rows_v, sem).wait()
        pltpu.sync_copy(res_hbm.at[pl.ds(base, bpw)], res_v)

        def row(i, carry):
            for j in range(D // 16):
                s = pl.ds(j * 16, 16)
                res_v[i, s] = res_v[i, s] - rows_v[i, s]
            return carry

        lax.fori_loop(0, bpw, row, 0)
        pltpu.sync_copy(res_v, out_hbm.at[pl.ds(base, bpw)])

    f = pl.kernel(
        body,
        out_type=jax.ShapeDtypeStruct((N, D), jnp.float32),
        mesh=mesh,
        scratch_types=[pltpu.VMEM((bpw,), jnp.int32),
                       pltpu.VMEM((bpw, D), jnp.float32),
                       pltpu.VMEM((bpw, D), jnp.float32),
                       pltpu.SemaphoreType.DMA],
    )
    return f(table, idx, res)


def _sc_gather_sub_final(table, idx, res, x):
    N, D = res.shape
    info = plsc.get_sparse_core_info()
    NC, NS = info.num_cores, info.num_subcores
    bpw = N // (NC * NS)
    mesh = plsc.VectorSubcoreMesh(core_axis_name="c", subcore_axis_name="s")

    def body(table_hbm, idx_hbm, res_hbm, x_hbm, out_hbm,
             idx_v, rows_v, res_v, x_v, sem):
        wid = lax.axis_index("s") * NC + lax.axis_index("c")
        base = wid * bpw
        pltpu.sync_copy(idx_hbm.at[pl.ds(base, bpw)], idx_v)
        pltpu.async_copy(table_hbm.at[idx_v], rows_v, sem).wait()
        pltpu.sync_copy(res_hbm.at[pl.ds(base, bpw)], res_v)
        pltpu.sync_copy(x_hbm.at[pl.ds(base, bpw)], x_v)

        def row(i, carry):
            for j in range(D // 16):
                s = pl.ds(j * 16, 16)
                x_v[i, s] = x_v[i, s] - res_v[i, s] + rows_v[i, s]
            return carry

        lax.fori_loop(0, bpw, row, 0)
        pltpu.sync_copy(x_v, out_hbm.at[pl.ds(base, bpw)])

    f = pl.kernel(
        body,
        out_type=jax.ShapeDtypeStruct((N, D), jnp.float32),
        mesh=mesh,
        scratch_types=[pltpu.VMEM((bpw,), jnp.int32),
                       pltpu.VMEM((bpw, D), jnp.float32),
                       pltpu.VMEM((bpw, D), jnp.float32),
                       pltpu.VMEM((bpw, D), jnp.float32),
                       pltpu.SemaphoreType.DMA],
    )
    return f(table, idx, res, x)


def kernel(input, codebooks):
    B, D, T = input.shape
    L, K, _ = codebooks.shape
    x = jnp.transpose(input, (0, 2, 1)).reshape(B * T, D)
    cb2_bf = (codebooks * -2.0).astype(jnp.bfloat16)
    wsq_all = jnp.sum(codebooks * codebooks, axis=-1)
    kio = jnp.arange(K, dtype=jnp.float32)
    res = x
    idx_list = []
    out_flat = None
    for l in range(L):
        rsq = jnp.sum(res * res, axis=-1)
        idx3 = _tc_argmin(res, rsq, cb2_bf[l], wsq_all[l], kio)
        idx_flat = idx3.reshape(-1)
        idx_list.append(idx3.reshape(B, T))
        if l + 1 < L:
            res = _sc_gather_sub(codebooks[l], idx_flat, res)
        else:
            out_flat = _sc_gather_sub_final(codebooks[l], idx_flat, res, x)
    out = out_flat.reshape(B, T, D).transpose(0, 2, 1).reshape(input.shape)
    indices = jnp.stack(idx_list, axis=1).reshape((B, L, T))
    return out, indices


# speedup vs baseline: 2.2431x; 1.0060x over previous
import functools

import jax
import jax.numpy as jnp
from jax import lax
from jax.experimental import pallas as pl
from jax.experimental.pallas import tpu as pltpu
from jax.experimental.pallas import tpu_sc as plsc

KT = 512
TT = 256


SK = 128


TPS = 2


def _argmin_body(w_ref, wsq_ref, kio_ref, r_ref, rsq_ref, idx_ref, *, n_s):
    r_bfs = [r_ref[pl.ds(j * TT, TT), :].astype(jnp.bfloat16)
             for j in range(TPS)]
    r2s = [rsq_ref[j] for j in range(TPS)]
    m8 = [None] * TPS
    i8 = [None] * TPS
    for s in range(n_s):
        w2 = w_ref[pl.ds(s * SK, SK), :]
        wsq_c = wsq_ref[pl.ds(s * SK, SK), :]
        io_c = kio_ref[pl.ds(s * SK, SK), :]
        for j in range(TPS):
            mcross = lax.dot_general(
                w2, r_bfs[j], (((1,), (1,)), ((), ())),
                preferred_element_type=jnp.float32)
            d2 = (r2s[j] + mcross) + wsq_c
            io = io_c
            v = d2
            n = SK
            while n > 8:
                h = n // 2
                a, b = v[:h], v[h:]
                lt = b < a
                io = jnp.where(lt, io[h:], io[:h])
                v = jnp.minimum(a, b)
                n = h
            if m8[j] is None:
                m8[j], i8[j] = v, io
            else:
                lt = v < m8[j]
                i8[j] = jnp.where(lt, io, i8[j])
                m8[j] = jnp.minimum(m8[j], v)
    for j in range(TPS):
        m = jnp.min(m8[j], axis=0)[None, :]
        ii = jnp.min(jnp.where(m8[j] == m, i8[j], jnp.float32(2**24)),
                     axis=0)[None, :]
        idx_ref[j] = ii.astype(jnp.int32).reshape(1, TT)


def _tc_argmin(res, rsq, w2_bf, wsq, kio):
    N, D = res.shape
    K = w2_bf.shape[0]
    n_t = N // TT
    n_g = n_t // TPS
    return pl.pallas_call(
        functools.partial(_argmin_body, n_s=K // SK),
        grid=(n_g,),
        in_specs=[
            pl.BlockSpec((K, D), lambda t: (0, 0)),
            pl.BlockSpec((K, 1), lambda t: (0, 0)),
            pl.BlockSpec((K, 1), lambda t: (0, 0)),
            pl.BlockSpec((TPS * TT, D), lambda t: (t, 0)),
            pl.BlockSpec((TPS, 1, TT), lambda t: (t, 0, 0)),
        ],
        out_specs=pl.BlockSpec((TPS, 1, TT), lambda t: (t, 0, 0)),
        out_shape=jax.ShapeDtypeStruct((n_t, 1, TT), jnp.int32),
    )(w2_bf, wsq.reshape(K, 1), kio.reshape(K, 1), res,
      rsq.reshape(n_t, 1, TT))


def _sc_gather_sub(table, idx, res):
    N, D = res.shape
    info = plsc.get_sparse_core_info()
    NC, NS = info.num_cores, info.num_subcores
    bpw = N // (NC * NS)
    mesh = plsc.VectorSubcoreMesh(core_axis_name="c", subcore_axis_name="s")

    def body(table_hbm, idx_hbm, res_hbm, out_hbm, idx_v, rows_v, res_v, sem):
        wid = lax.axis_index("s") * NC + lax.axis_index("c")
        base = wid * bpw
        pltpu.sync_copy(idx_hbm.at[pl.ds(base, bpw)], idx_v)
        pltpu.async_copy(table_hbm.at[idx_v], rows_v, sem).wait()
        pltpu.sync_copy(res_hbm.at[pl.ds(base, bpw)], res_v)

        @plsc.parallel_loop(0, bpw, unroll=2)
        def _row(i):
            for j in range(D // 16):
                s = pl.ds(j * 16, 16)
                res_v[i, s] = res_v[i, s] - rows_v[i, s]

        pltpu.sync_copy(res_v, out_hbm.at[pl.ds(base, bpw)])

    f = pl.kernel(
        body,
        out_type=jax.ShapeDtypeStruct((N, D), jnp.float32),
        mesh=mesh,
        scratch_types=[pltpu.VMEM((bpw,), jnp.int32),
                       pltpu.VMEM((bpw, D), jnp.float32),
                       pltpu.VMEM((bpw, D), jnp.float32),
                       pltpu.SemaphoreType.DMA],
    )
    return f(table, idx, res)


def _sc_gather_sub_final(table, idx, res, x):
    N, D = res.shape
    info = plsc.get_sparse_core_info()
    NC, NS = info.num_cores, info.num_subcores
    bpw = N // (NC * NS)
    mesh = plsc.VectorSubcoreMesh(core_axis_name="c", subcore_axis_name="s")

    def body(table_hbm, idx_hbm, res_hbm, x_hbm, out_hbm,
             idx_v, rows_v, res_v, x_v, sem):
        wid = lax.axis_index("s") * NC + lax.axis_index("c")
        base = wid * bpw
        pltpu.sync_copy(idx_hbm.at[pl.ds(base, bpw)], idx_v)
        pltpu.async_copy(table_hbm.at[idx_v], rows_v, sem).wait()
        pltpu.sync_copy(res_hbm.at[pl.ds(base, bpw)], res_v)
        pltpu.sync_copy(x_hbm.at[pl.ds(base, bpw)], x_v)

        @plsc.parallel_loop(0, bpw, unroll=2)
        def _row(i):
            for j in range(D // 16):
                s = pl.ds(j * 16, 16)
                x_v[i, s] = x_v[i, s] - res_v[i, s] + rows_v[i, s]

        pltpu.sync_copy(x_v, out_hbm.at[pl.ds(base, bpw)])

    f = pl.kernel(
        body,
        out_type=jax.ShapeDtypeStruct((N, D), jnp.float32),
        mesh=mesh,
        scratch_types=[pltpu.VMEM((bpw,), jnp.int32),
                       pltpu.VMEM((bpw, D), jnp.float32),
                       pltpu.VMEM((bpw, D), jnp.float32),
                       pltpu.VMEM((bpw, D), jnp.float32),
                       pltpu.SemaphoreType.DMA],
    )
    return f(table, idx, res, x)


def kernel(input, codebooks):
    B, D, T = input.shape
    L, K, _ = codebooks.shape
    x = jnp.transpose(input, (0, 2, 1)).reshape(B * T, D)
    cb2_bf = (codebooks * -2.0).astype(jnp.bfloat16)
    wsq_all = jnp.sum(codebooks * codebooks, axis=-1)
    kio = jnp.arange(K, dtype=jnp.float32)
    res = x
    idx_list = []
    out_flat = None
    for l in range(L):
        rsq = jnp.sum(res * res, axis=-1)
        idx3 = _tc_argmin(res, rsq, cb2_bf[l], wsq_all[l], kio)
        idx_flat = idx3.reshape(-1)
        idx_list.append(idx3.reshape(B, T))
        if l + 1 < L:
            res = _sc_gather_sub(codebooks[l], idx_flat, res)
        else:
            out_flat = _sc_gather_sub_final(codebooks[l], idx_flat, res, x)
    out = out_flat.reshape(B, T, D).transpose(0, 2, 1).reshape(input.shape)
    indices = jnp.stack(idx_list, axis=1).reshape((B, L, T))
    return out, indices


# speedup vs baseline: 2.2868x; 1.0195x over previous
import functools

import jax
import jax.numpy as jnp
from jax import lax
from jax.experimental import pallas as pl
from jax.experimental.pallas import tpu as pltpu
from jax.experimental.pallas import tpu_sc as plsc

KT = 512
TT = 256


SK = 128


TPS = 4


def _argmin_body(w_ref, wsq_ref, kio_ref, r_ref, rsq_ref, idx_ref, *, n_s):
    r_bfs = [r_ref[pl.ds(j * TT, TT), :].astype(jnp.bfloat16)
             for j in range(TPS)]
    r2s = [rsq_ref[j] for j in range(TPS)]
    m8 = [None] * TPS
    i8 = [None] * TPS
    for s in range(n_s):
        w2 = w_ref[pl.ds(s * SK, SK), :]
        wsq_c = wsq_ref[pl.ds(s * SK, SK), :]
        io_c = kio_ref[pl.ds(s * SK, SK), :]
        for j in range(TPS):
            mcross = lax.dot_general(
                w2, r_bfs[j], (((1,), (1,)), ((), ())),
                preferred_element_type=jnp.float32)
            d2 = (r2s[j] + mcross) + wsq_c
            io = io_c
            v = d2
            n = SK
            while n > 8:
                h = n // 2
                a, b = v[:h], v[h:]
                lt = b < a
                io = jnp.where(lt, io[h:], io[:h])
                v = jnp.minimum(a, b)
                n = h
            if m8[j] is None:
                m8[j], i8[j] = v, io
            else:
                lt = v < m8[j]
                i8[j] = jnp.where(lt, io, i8[j])
                m8[j] = jnp.minimum(m8[j], v)
    for j in range(TPS):
        m = jnp.min(m8[j], axis=0)[None, :]
        ii = jnp.min(jnp.where(m8[j] == m, i8[j], jnp.float32(2**24)),
                     axis=0)[None, :]
        idx_ref[j] = ii.astype(jnp.int32).reshape(1, TT)


def _tc_argmin(res, rsq, w2_bf, wsq, kio):
    N, D = res.shape
    K = w2_bf.shape[0]
    n_t = N // TT
    n_g = n_t // TPS
    return pl.pallas_call(
        functools.partial(_argmin_body, n_s=K // SK),
        grid=(n_g,),
        in_specs=[
            pl.BlockSpec((K, D), lambda t: (0, 0)),
            pl.BlockSpec((K, 1), lambda t: (0, 0)),
            pl.BlockSpec((K, 1), lambda t: (0, 0)),
            pl.BlockSpec((TPS * TT, D), lambda t: (t, 0)),
            pl.BlockSpec((TPS, 1, TT), lambda t: (t, 0, 0)),
        ],
        out_specs=pl.BlockSpec((TPS, 1, TT), lambda t: (t, 0, 0)),
        out_shape=jax.ShapeDtypeStruct((n_t, 1, TT), jnp.int32),
    )(w2_bf, wsq.reshape(K, 1), kio.reshape(K, 1), res,
      rsq.reshape(n_t, 1, TT))


def _sc_gather_sub(table, idx, res):
    N, D = res.shape
    info = plsc.get_sparse_core_info()
    NC, NS = info.num_cores, info.num_subcores
    bpw = N // (NC * NS)
    mesh = plsc.VectorSubcoreMesh(core_axis_name="c", subcore_axis_name="s")

    def body(table_hbm, idx_hbm, res_hbm, out_hbm, idx_v, rows_v, res_v,
             sem, sem2):
        wid = lax.axis_index("s") * NC + lax.axis_index("c")
        base = wid * bpw
        pltpu.sync_copy(idx_hbm.at[pl.ds(base, bpw)], idx_v)
        cp_r = pltpu.async_copy(res_hbm.at[pl.ds(base, bpw)], res_v, sem2)
        pltpu.async_copy(table_hbm.at[idx_v], rows_v, sem).wait()
        cp_r.wait()

        @plsc.parallel_loop(0, bpw, unroll=2)
        def _row(i):
            for j in range(D // 16):
                s = pl.ds(j * 16, 16)
                res_v[i, s] = res_v[i, s] - rows_v[i, s]

        pltpu.sync_copy(res_v, out_hbm.at[pl.ds(base, bpw)])

    f = pl.kernel(
        body,
        out_type=jax.ShapeDtypeStruct((N, D), jnp.float32),
        mesh=mesh,
        scratch_types=[pltpu.VMEM((bpw,), jnp.int32),
                       pltpu.VMEM((bpw, D), jnp.float32),
                       pltpu.VMEM((bpw, D), jnp.float32),
                       pltpu.SemaphoreType.DMA,
                       pltpu.SemaphoreType.DMA],
    )
    return f(table, idx, res)


def _sc_gather_sub_final(table, idx, res, x):
    N, D = res.shape
    info = plsc.get_sparse_core_info()
    NC, NS = info.num_cores, info.num_subcores
    bpw = N // (NC * NS)
    mesh = plsc.VectorSubcoreMesh(core_axis_name="c", subcore_axis_name="s")

    def body(table_hbm, idx_hbm, res_hbm, x_hbm, out_hbm,
             idx_v, rows_v, res_v, x_v, sem, sem2, sem3):
        wid = lax.axis_index("s") * NC + lax.axis_index("c")
        base = wid * bpw
        pltpu.sync_copy(idx_hbm.at[pl.ds(base, bpw)], idx_v)
        cp_r = pltpu.async_copy(res_hbm.at[pl.ds(base, bpw)], res_v, sem2)
        cp_x = pltpu.async_copy(x_hbm.at[pl.ds(base, bpw)], x_v, sem3)
        pltpu.async_copy(table_hbm.at[idx_v], rows_v, sem).wait()
        cp_r.wait()
        cp_x.wait()

        @plsc.parallel_loop(0, bpw, unroll=2)
        def _row(i):
            for j in range(D // 16):
                s = pl.ds(j * 16, 16)
                x_v[i, s] = x_v[i, s] - res_v[i, s] + rows_v[i, s]

        pltpu.sync_copy(x_v, out_hbm.at[pl.ds(base, bpw)])

    f = pl.kernel(
        body,
        out_type=jax.ShapeDtypeStruct((N, D), jnp.float32),
        mesh=mesh,
        scratch_types=[pltpu.VMEM((bpw,), jnp.int32),
                       pltpu.VMEM((bpw, D), jnp.float32),
                       pltpu.VMEM((bpw, D), jnp.float32),
                       pltpu.VMEM((bpw, D), jnp.float32),
                       pltpu.SemaphoreType.DMA,
                       pltpu.SemaphoreType.DMA,
                       pltpu.SemaphoreType.DMA],
    )
    return f(table, idx, res, x)


def kernel(input, codebooks):
    B, D, T = input.shape
    L, K, _ = codebooks.shape
    x = jnp.transpose(input, (0, 2, 1)).reshape(B * T, D)
    cb2_bf = (codebooks * -2.0).astype(jnp.bfloat16)
    wsq_all = jnp.sum(codebooks * codebooks, axis=-1)
    kio = jnp.arange(K, dtype=jnp.float32)
    res = x
    idx_list = []
    out_flat = None
    for l in range(L):
        rsq = jnp.sum(res * res, axis=-1)
        idx3 = _tc_argmin(res, rsq, cb2_bf[l], wsq_all[l], kio)
        idx_flat = idx3.reshape(-1)
        idx_list.append(idx3.reshape(B, T))
        if l + 1 < L:
            res = _sc_gather_sub(codebooks[l], idx_flat, res)
        else:
            out_flat = _sc_gather_sub_final(codebooks[l], idx_flat, res, x)
    out = out_flat.reshape(B, T, D).transpose(0, 2, 1).reshape(input.shape)
    indices = jnp.stack(idx_list, axis=1).reshape((B, L, T))
    return out, indices
